# baseline (device time: 27252 ns/iter reference)
import jax
import jax.numpy as jnp
from jax import lax
from jax.experimental import pallas as pl
from jax.experimental.pallas import tpu as pltpu

N_DEV = 32
E_PER = 2
N_TOK = 512
ROWS_PER = N_TOK // N_DEV


def kernel(x, router_W, route_idx, expert_W, shared_W):
    n_tok, d_model = x.shape
    d_out = shared_W.shape[1]

    def body(x_ref, router_ref, ridx_ref, ew_ref, sw_ref, out_ref,
             contrib_ref, recv_ref, send_sems, recv_sems):
        my = lax.axis_index("i")

        barrier_sem = pltpu.get_barrier_semaphore()
        for d in range(1, N_DEV):
            peer = lax.rem(my + d, N_DEV)
            pl.semaphore_signal(
                barrier_sem, inc=1,
                device_id=(peer,), device_id_type=pl.DeviceIdType.MESH,
            )
        pl.semaphore_wait(barrier_sem, N_DEV - 1)

        xv = x_ref[:, :]
        scores = jnp.dot(xv, router_ref[:, :],
                         preferred_element_type=jnp.float32)
        s_max = jnp.max(scores, axis=-1, keepdims=True)
        p = jnp.exp(scores - s_max)
        probs = p / jnp.sum(p, axis=-1, keepdims=True)

        xb = xv.astype(jnp.bfloat16)
        ridx = ridx_ref[:, :]
        eidx = lax.broadcasted_iota(jnp.int32, scores.shape, 1)
        contrib = jnp.zeros((n_tok, d_out), jnp.float32)
        for e in range(E_PER):
            e_g = my * E_PER + e
            y = jnp.dot(xb, ew_ref[e].astype(jnp.bfloat16),
                        preferred_element_type=jnp.float32)
            p_e = jnp.sum(jnp.where(eidx == e_g, probs, 0.0),
                          axis=1, keepdims=True)
            gate = jnp.where(ridx == e_g, p_e, 0.0)
            contrib = contrib + gate * y
        contrib_ref[:, :] = contrib

        descs = []
        for d in range(1, N_DEV):
            dst = lax.rem(my + d, N_DEV)
            rdma = pltpu.make_async_remote_copy(
                src_ref=contrib_ref.at[pl.ds(dst * ROWS_PER, ROWS_PER), :],
                dst_ref=recv_ref.at[pl.ds(my * ROWS_PER, ROWS_PER), :],
                send_sem=send_sems.at[d - 1],
                recv_sem=recv_sems.at[d - 1],
                device_id=(dst,),
                device_id_type=pl.DeviceIdType.MESH,
            )
            rdma.start()
            descs.append(rdma)

        recv_ref[pl.ds(my * ROWS_PER, ROWS_PER), :] = (
            contrib_ref[pl.ds(my * ROWS_PER, ROWS_PER), :])

        xrows = x_ref[pl.ds(my * ROWS_PER, ROWS_PER), :]
        sh = jnp.dot(xrows.astype(jnp.bfloat16),
                     sw_ref[:, :].astype(jnp.bfloat16),
                     preferred_element_type=jnp.float32)

        for rdma in descs:
            rdma.wait_recv()

        total = sh
        for j in range(N_DEV):
            total = total + recv_ref[j * ROWS_PER:(j + 1) * ROWS_PER, :]
        out_ref[:, :] = total

        for rdma in descs:
            rdma.wait_send()

    return pl.pallas_call(
        body,
        out_shape=jax.ShapeDtypeStruct((ROWS_PER, d_out), jnp.float32),
        in_specs=[pl.BlockSpec(memory_space=pltpu.VMEM)] * 5,
        out_specs=pl.BlockSpec(memory_space=pltpu.VMEM),
        scratch_shapes=[
            pltpu.VMEM((n_tok, d_out), jnp.float32),
            pltpu.VMEM((n_tok, d_out), jnp.float32),
            pltpu.SemaphoreType.DMA((N_DEV - 1,)),
            pltpu.SemaphoreType.DMA((N_DEV - 1,)),
        ],
        compiler_params=pltpu.CompilerParams(collective_id=0),
    )(x, router_W, route_idx, expert_W, shared_W)


# device time: 20941 ns/iter; 1.3014x vs baseline; 1.3014x over previous
import jax
import jax.numpy as jnp
from jax import lax
from jax.experimental import pallas as pl
from jax.experimental.pallas import tpu as pltpu

N_DEV = 32
E_PER = 2
N_TOK = 512
ROWS_PER = N_TOK // N_DEV


def kernel(x, router_W, route_idx, expert_W, shared_W):
    n_tok, d_model = x.shape
    d_out = shared_W.shape[1]

    def body(x_ref, router_ref, ridx_ref, ew_ref, sw_ref, out_ref,
             contrib_ref, recv_ref, send_sems, recv_sems):
        my = lax.axis_index("i")

        barrier_sem = pltpu.get_barrier_semaphore()
        for d in range(1, N_DEV):
            peer = lax.rem(my + d, N_DEV)
            pl.semaphore_signal(
                barrier_sem, inc=1,
                device_id=(peer,), device_id_type=pl.DeviceIdType.MESH,
            )

        xv = x_ref[:, :]
        scores = jnp.dot(xv, router_ref[:, :],
                         preferred_element_type=jnp.float32)
        s_max = jnp.max(scores, axis=-1, keepdims=True)
        p = jnp.exp(scores - s_max)
        probs = p / jnp.sum(p, axis=-1, keepdims=True)

        xb = xv.astype(jnp.bfloat16)
        ridx = ridx_ref[:, :]
        eidx = lax.broadcasted_iota(jnp.int32, scores.shape, 1)
        contrib = jnp.zeros((n_tok, d_out), jnp.float32)
        for e in range(E_PER):
            e_g = my * E_PER + e
            y = jnp.dot(xb, ew_ref[e].astype(jnp.bfloat16),
                        preferred_element_type=jnp.float32)
            p_e = jnp.sum(jnp.where(eidx == e_g, probs, 0.0),
                          axis=1, keepdims=True)
            gate = jnp.where(ridx == e_g, p_e, 0.0)
            contrib = contrib + gate * y
        contrib_ref[:, :] = contrib.astype(jnp.bfloat16)

        pl.semaphore_wait(barrier_sem, N_DEV - 1)

        descs = []
        for d in range(1, N_DEV):
            dst = lax.rem(my + d, N_DEV)
            rdma = pltpu.make_async_remote_copy(
                src_ref=contrib_ref.at[pl.ds(dst * ROWS_PER, ROWS_PER), :],
                dst_ref=recv_ref.at[pl.ds(my * ROWS_PER, ROWS_PER), :],
                send_sem=send_sems.at[d - 1],
                recv_sem=recv_sems.at[d - 1],
                device_id=(dst,),
                device_id_type=pl.DeviceIdType.MESH,
            )
            rdma.start()
            descs.append(rdma)

        recv_ref[pl.ds(my * ROWS_PER, ROWS_PER), :] = (
            contrib_ref[pl.ds(my * ROWS_PER, ROWS_PER), :])

        xrows = x_ref[pl.ds(my * ROWS_PER, ROWS_PER), :]
        sh = jnp.dot(xrows.astype(jnp.bfloat16),
                     sw_ref[:, :].astype(jnp.bfloat16),
                     preferred_element_type=jnp.float32)

        for rdma in descs:
            rdma.wait_recv()

        total = sh
        for j in range(N_DEV):
            total = total + recv_ref[j * ROWS_PER:(j + 1) * ROWS_PER, :].astype(
                jnp.float32)
        out_ref[:, :] = total

        for rdma in descs:
            rdma.wait_send()

    return pl.pallas_call(
        body,
        out_shape=jax.ShapeDtypeStruct((ROWS_PER, d_out), jnp.float32),
        in_specs=[pl.BlockSpec(memory_space=pltpu.VMEM)] * 5,
        out_specs=pl.BlockSpec(memory_space=pltpu.VMEM),
        scratch_shapes=[
            pltpu.VMEM((n_tok, d_out), jnp.bfloat16),
            pltpu.VMEM((n_tok, d_out), jnp.bfloat16),
            pltpu.SemaphoreType.DMA((N_DEV - 1,)),
            pltpu.SemaphoreType.DMA((N_DEV - 1,)),
        ],
        compiler_params=pltpu.CompilerParams(collective_id=0),
    )(x, router_W, route_idx, expert_W, shared_W)
